# Initial kernel scaffold; baseline (speedup 1.0000x reference)
import jax
import jax.numpy as jnp
from jax.experimental import pallas as pl


def kernel(x, edge_index, edge_attr, batch, params):
    raise NotImplementedError("write your pallas kernel here")


# speedup vs baseline: 1.0066x; 1.0066x over previous
import functools

import jax
import jax.numpy as jnp
from jax.experimental import pallas as pl

N_NODES = 10000
NUM_GRAPHS = 64
EPS = 1e-7


def _gen_conv_jnp(x, src, dst, e, p):
    m = jax.nn.relu(x[src] + e) + EPS
    mmax = jax.ops.segment_max(m, dst, num_segments=N_NODES)
    mmax = jnp.where(jnp.isfinite(mmax), mmax, 0.0)
    ex = jnp.exp(m - mmax[dst])
    den = jax.ops.segment_sum(ex, dst, num_segments=N_NODES)
    alpha = ex / (den[dst] + 1e-16)
    agg = jax.ops.segment_sum(alpha * m, dst, num_segments=N_NODES)
    h = agg + x
    h = jax.nn.relu(h @ p['W1'] + p['b1'])
    return h @ p['W2'] + p['b2']


def _tail_kernel(h_ref, batch_ref, w1_ref, b1_ref, w2_ref, b2_ref,
                 w3_ref, b3_ref, out_ref):
    h = h_ref[...]
    batch = batch_ref[...]
    gids = jax.lax.broadcasted_iota(jnp.int32, (NUM_GRAPHS, N_NODES), 0)
    mask = (batch == gids).astype(jnp.float32)
    counts = jnp.sum(mask, axis=1, keepdims=True)
    pooled = jnp.dot(mask, h, preferred_element_type=jnp.float32)
    pooled = pooled / jnp.maximum(counts, 1.0)
    t = pooled @ w1_ref[...] + b1_ref[...]
    t = t @ w2_ref[...] + b2_ref[...]
    t = t @ w3_ref[...] + b3_ref[...]
    t = t - jnp.max(t, axis=1, keepdims=True)
    lse = jnp.log(jnp.sum(jnp.exp(t), axis=1, keepdims=True))
    out_ref[...] = t - lse


def kernel(x, edge_index, edge_attr, batch, params):
    src = edge_index[0]
    dst = edge_index[1]
    h = x
    for i in range(5):
        p = params['conv%d' % i]
        e = edge_attr @ p['We'] + p['be']
        h = jax.nn.relu(_gen_conv_jnp(h, src, dst, e, p))

    out = pl.pallas_call(
        _tail_kernel,
        out_shape=jax.ShapeDtypeStruct((NUM_GRAPHS, 10), jnp.float32),
    )(h, batch.reshape(1, N_NODES),
      params['dense1']['W'], params['dense1']['b'].reshape(1, -1),
      params['dense2']['W'], params['dense2']['b'].reshape(1, -1),
      params['dense3']['W'], params['dense3']['b'].reshape(1, -1))
    return out


# speedup vs baseline: 1.1631x; 1.1555x over previous
import jax
import jax.numpy as jnp
from jax.experimental import pallas as pl

N_NODES = 10000
N_EDGES = 320000
NUM_GRAPHS = 64
EPS = 1e-7

_CONV = [(128, 64), (64, 64), (64, 128), (128, 256), (256, 512)]


def _edge_mm(ea, We, be):
    RB = 2000
    grid = N_EDGES // RB
    ic = We.shape[1]

    def kern(ea_ref, we_ref, be_ref, out_ref):
        out_ref[...] = (jnp.dot(ea_ref[...], we_ref[...],
                                preferred_element_type=jnp.float32)
                        + be_ref[...])

    return pl.pallas_call(
        kern,
        grid=(grid,),
        in_specs=[
            pl.BlockSpec((RB, 16), lambda i: (i, 0)),
            pl.BlockSpec((16, ic), lambda i: (0, 0)),
            pl.BlockSpec((1, ic), lambda i: (0, 0)),
        ],
        out_specs=pl.BlockSpec((RB, ic), lambda i: (i, 0)),
        out_shape=jax.ShapeDtypeStruct((N_EDGES, ic), jnp.float32),
    )(ea, We, be.reshape(1, ic))


def _node_mlp(seg, h, W1, b1, W2, b2):
    NB = 2000
    grid = N_NODES // NB
    ic = W1.shape[0]
    hid = W1.shape[1]
    oc = W2.shape[1]

    def kern(seg_ref, h_ref, w1_ref, b1_ref, w2_ref, b2_ref, out_ref):
        s = seg_ref[...]
        den = s[:, :ic]
        wsum = s[:, ic:]
        hin = wsum / (den + 1e-16) + h_ref[...]
        t = jax.nn.relu(jnp.dot(hin, w1_ref[...],
                                preferred_element_type=jnp.float32)
                        + b1_ref[...])
        out_ref[...] = jax.nn.relu(
            jnp.dot(t, w2_ref[...], preferred_element_type=jnp.float32)
            + b2_ref[...])

    return pl.pallas_call(
        kern,
        grid=(grid,),
        in_specs=[
            pl.BlockSpec((NB, 2 * ic), lambda i: (i, 0)),
            pl.BlockSpec((NB, ic), lambda i: (i, 0)),
            pl.BlockSpec((ic, hid), lambda i: (0, 0)),
            pl.BlockSpec((1, hid), lambda i: (0, 0)),
            pl.BlockSpec((hid, oc), lambda i: (0, 0)),
            pl.BlockSpec((1, oc), lambda i: (0, 0)),
        ],
        out_specs=pl.BlockSpec((NB, oc), lambda i: (i, 0)),
        out_shape=jax.ShapeDtypeStruct((N_NODES, oc), jnp.float32),
    )(seg, h, W1, b1.reshape(1, hid), W2, b2.reshape(1, oc))


def _tail_kernel(h_ref, batch_ref, w1_ref, b1_ref, w2_ref, b2_ref,
                 w3_ref, b3_ref, out_ref):
    h = h_ref[...]
    batch = batch_ref[...]
    gids = jax.lax.broadcasted_iota(jnp.int32, (NUM_GRAPHS, N_NODES), 0)
    mask = (batch == gids).astype(jnp.float32)
    counts = jnp.sum(mask, axis=1, keepdims=True)
    pooled = jnp.dot(mask, h, preferred_element_type=jnp.float32)
    pooled = pooled / jnp.maximum(counts, 1.0)
    t = pooled @ w1_ref[...] + b1_ref[...]
    t = t @ w2_ref[...] + b2_ref[...]
    t = t @ w3_ref[...] + b3_ref[...]
    t = t - jnp.max(t, axis=1, keepdims=True)
    lse = jnp.log(jnp.sum(jnp.exp(t), axis=1, keepdims=True))
    out_ref[...] = t - lse


def kernel(x, edge_index, edge_attr, batch, params):
    src = edge_index[0]
    dst = edge_index[1]
    h = x
    for i, (ic, oc) in enumerate(_CONV):
        p = params['conv%d' % i]
        e = _edge_mm(edge_attr, p['We'], p['be'])
        m = jax.nn.relu(h[src] + e) + EPS
        mmax = jax.ops.segment_max(m, dst, num_segments=N_NODES)
        mmax = jnp.where(jnp.isfinite(mmax), mmax, 0.0)
        ex = jnp.exp(m - mmax[dst])
        den = jax.ops.segment_sum(ex, dst, num_segments=N_NODES)
        wsum = jax.ops.segment_sum(ex * m, dst, num_segments=N_NODES)
        seg = jnp.concatenate([den, wsum], axis=1)
        h = _node_mlp(seg, h, p['W1'], p['b1'], p['W2'], p['b2'])

    out = pl.pallas_call(
        _tail_kernel,
        out_shape=jax.ShapeDtypeStruct((NUM_GRAPHS, 10), jnp.float32),
    )(h, batch.reshape(1, N_NODES),
      params['dense1']['W'], params['dense1']['b'].reshape(1, -1),
      params['dense2']['W'], params['dense2']['b'].reshape(1, -1),
      params['dense3']['W'], params['dense3']['b'].reshape(1, -1))
    return out


# speedup vs baseline: 1.1939x; 1.0265x over previous
import jax
import jax.numpy as jnp
from jax.experimental import pallas as pl

N_NODES = 10000
N_EDGES = 320000
NUM_GRAPHS = 64
EPS = 1e-7

_CONV = [(128, 64), (64, 64), (64, 128), (128, 256), (256, 512)]


def _edge_mm(ea, We, be):
    RB = 2000
    grid = N_EDGES // RB
    ic = We.shape[1]

    def kern(ea_ref, we_ref, be_ref, out_ref):
        out_ref[...] = (jnp.dot(ea_ref[...], we_ref[...],
                                preferred_element_type=jnp.float32)
                        + be_ref[...])

    return pl.pallas_call(
        kern,
        grid=(grid,),
        in_specs=[
            pl.BlockSpec((RB, 16), lambda i: (i, 0)),
            pl.BlockSpec((16, ic), lambda i: (0, 0)),
            pl.BlockSpec((1, ic), lambda i: (0, 0)),
        ],
        out_specs=pl.BlockSpec((RB, ic), lambda i: (i, 0)),
        out_shape=jax.ShapeDtypeStruct((N_EDGES, ic), jnp.float32),
    )(ea, We, be.reshape(1, ic))


def _node_mlp(seg, h, W1, b1, W2, b2):
    NB = 2000
    grid = N_NODES // NB
    ic = W1.shape[0]
    hid = W1.shape[1]
    oc = W2.shape[1]

    def kern(seg_ref, h_ref, w1_ref, b1_ref, w2_ref, b2_ref, out_ref):
        s = seg_ref[...]
        den = s[:, :ic]
        wsum = s[:, ic:]
        hin = wsum / (den + 1e-16) + h_ref[...]
        t = jax.nn.relu(jnp.dot(hin, w1_ref[...],
                                preferred_element_type=jnp.float32)
                        + b1_ref[...])
        out_ref[...] = jax.nn.relu(
            jnp.dot(t, w2_ref[...], preferred_element_type=jnp.float32)
            + b2_ref[...])

    return pl.pallas_call(
        kern,
        grid=(grid,),
        in_specs=[
            pl.BlockSpec((NB, 2 * ic), lambda i: (i, 0)),
            pl.BlockSpec((NB, ic), lambda i: (i, 0)),
            pl.BlockSpec((ic, hid), lambda i: (0, 0)),
            pl.BlockSpec((1, hid), lambda i: (0, 0)),
            pl.BlockSpec((hid, oc), lambda i: (0, 0)),
            pl.BlockSpec((1, oc), lambda i: (0, 0)),
        ],
        out_specs=pl.BlockSpec((NB, oc), lambda i: (i, 0)),
        out_shape=jax.ShapeDtypeStruct((N_NODES, oc), jnp.float32),
    )(seg, h, W1, b1.reshape(1, hid), W2, b2.reshape(1, oc))


def _tail_kernel(h_ref, batch_ref, w1_ref, b1_ref, w2_ref, b2_ref,
                 w3_ref, b3_ref, out_ref):
    h = h_ref[...]
    batch = batch_ref[...]
    gids = jax.lax.broadcasted_iota(jnp.int32, (NUM_GRAPHS, N_NODES), 0)
    mask = (batch == gids).astype(jnp.float32)
    counts = jnp.sum(mask, axis=1, keepdims=True)
    pooled = jnp.dot(mask, h, preferred_element_type=jnp.float32)
    pooled = pooled / jnp.maximum(counts, 1.0)
    t = pooled @ w1_ref[...] + b1_ref[...]
    t = t @ w2_ref[...] + b2_ref[...]
    t = t @ w3_ref[...] + b3_ref[...]
    t = t - jnp.max(t, axis=1, keepdims=True)
    lse = jnp.log(jnp.sum(jnp.exp(t), axis=1, keepdims=True))
    out_ref[...] = t - lse


def kernel(x, edge_index, edge_attr, batch, params):
    src = edge_index[0]
    dst = edge_index[1]
    h = x
    for i, (ic, oc) in enumerate(_CONV):
        p = params['conv%d' % i]
        e = _edge_mm(edge_attr, p['We'], p['be'])
        m = jax.nn.relu(h[src] + e) + EPS
        mmax = jax.ops.segment_max(m, dst, num_segments=N_NODES)
        mmax = jnp.where(jnp.isfinite(mmax), mmax, 0.0)
        ex = jnp.exp(m - mmax[dst])
        seg = jax.ops.segment_sum(
            jnp.concatenate([ex, ex * m], axis=1), dst,
            num_segments=N_NODES)
        h = _node_mlp(seg, h, p['W1'], p['b1'], p['W2'], p['b2'])

    out = pl.pallas_call(
        _tail_kernel,
        out_shape=jax.ShapeDtypeStruct((NUM_GRAPHS, 10), jnp.float32),
    )(h, batch.reshape(1, N_NODES),
      params['dense1']['W'], params['dense1']['b'].reshape(1, -1),
      params['dense2']['W'], params['dense2']['b'].reshape(1, -1),
      params['dense3']['W'], params['dense3']['b'].reshape(1, -1))
    return out
